# baseline (speedup 1.0000x reference)
import functools

import jax
import jax.numpy as jnp
from jax import lax
from jax.experimental import pallas as pl
from jax.experimental.pallas import tpu as pltpu
from jax.experimental.pallas import tpu_sc as plsc

L = 16


def _floor_f32(x):
    t = x.astype(jnp.int32)
    return t - jnp.where(t.astype(jnp.float32) > x, 1, 0)


def _sampler_body(ppw, ksub, h, w, c, log2_p,
                  table_hbm, pts_hbm, out_hbm,
                  pts_v, iw, rbuf, out_v, gsems):
    num_cores = plsc.get_sparse_core_info().num_cores
    wid = lax.axis_index("s") * num_cores + lax.axis_index("c")
    pbase = wid * ppw
    pltpu.sync_copy(pts_hbm.at[pl.ds(2 * pbase, 2 * ppw)], pts_v)

    nsub = ppw // ksub
    lanes = lax.iota(jnp.int32, L)
    lanes2 = lanes * 2

    def stage(j, slot):
        i_v = iw[slot][0]
        w_v = iw[slot][1]
        for g in range(ksub // L):
            off = j * ksub + g * L
            px = plsc.load_gather(pts_v, [lanes2 + 2 * off])
            py = plsc.load_gather(pts_v, [lanes2 + (2 * off + 1)])
            gx = (2.0 * px - 1.0) + 1.0
            gy = (2.0 * py - 1.0) + 1.0
            x = (gx * w - 1.0) * 0.5
            y = (gy * h - 1.0) * 0.5
            x0 = _floor_f32(x)
            y0 = _floor_f32(y)
            wx1 = x - x0.astype(jnp.float32)
            wx0 = 1.0 - wx1
            wy1 = y - y0.astype(jnp.float32)
            wy0 = 1.0 - wy1
            x1 = x0 + 1
            y1 = y0 + 1
            fx0 = jnp.where((x0 >= 0) & (x0 <= w - 1), wx0, 0.0)
            fx1 = jnp.where((x1 >= 0) & (x1 <= w - 1), wx1, 0.0)
            fy0 = jnp.where((y0 >= 0) & (y0 <= h - 1), wy0, 0.0)
            fy1 = jnp.where((y1 >= 0) & (y1 <= h - 1), wy1, 0.0)
            cx0 = jnp.minimum(jnp.maximum(x0, 0), w - 1)
            cx1 = jnp.minimum(jnp.maximum(x1, 0), w - 1)
            cy0 = jnp.minimum(jnp.maximum(y0, 0), h - 1)
            cy1 = jnp.minimum(jnp.maximum(y1, 0), h - 1)
            gp = pbase + off + lanes
            tb = gp & jnp.int32(~(2 ** log2_p - 1))
            row0 = tb + cy0 * w
            row1 = tb + cy1 * w
            sl = pl.ds(g * L, L)
            i_v[0][sl] = row0 + cx0
            i_v[1][sl] = row0 + cx1
            i_v[2][sl] = row1 + cx0
            i_v[3][sl] = row1 + cx1
            w_v[0][sl] = fy0 * fx0
            w_v[1][sl] = fy0 * fx1
            w_v[2][sl] = fy1 * fx0
            w_v[3][sl] = fy1 * fx1
        for k in range(4):
            pltpu.async_copy(table_hbm.at[i_v[k]], rbuf[slot][k], gsems[slot])

    def finish(j, slot):
        i_v = iw[slot][0]
        w_v = iw[slot][1]
        for k in range(4):
            pltpu.make_async_copy(table_hbm.at[i_v[k]], rbuf[slot][k],
                                  gsems[slot]).wait()

        r00, r01, r10, r11 = rbuf[slot]
        for g in range(ksub // L):
            sl = pl.ds(g * L, L)
            a00 = w_v[0][sl]
            a01 = w_v[1][sl]
            a10 = w_v[2][sl]
            a11 = w_v[3][sl]
            rows = lanes + g * L

            @plsc.parallel_loop(0, c, unroll=2)
            def _combine(ch, rows=rows, a00=a00, a01=a01, a10=a10, a11=a11):
                col = jnp.full((L,), ch, jnp.int32)
                val = ((plsc.load_gather(r00, [rows, col]) * a00
                        + plsc.load_gather(r01, [rows, col]) * a01)
                       + (plsc.load_gather(r10, [rows, col]) * a10
                          + plsc.load_gather(r11, [rows, col]) * a11))
                plsc.store_scatter(out_v, [rows, col], val)
        pltpu.sync_copy(out_v, out_hbm.at[pl.ds(pbase + j * ksub, ksub)])

    stage(0, 0)

    def sub2(j2, _):
        j = j2 * 2
        stage(j + 1, 1)
        finish(j, 0)

        @pl.when(j + 2 < nsub)
        def _():
            stage(j + 2, 0)

        finish(j + 1, 1)
        return 0

    lax.fori_loop(0, nsub // 2, sub2, 0)


def kernel(sample_map, sample_pts):
    n, c, h, w = sample_map.shape
    _, p, _ = sample_pts.shape
    np_total = n * p
    assert p & (p - 1) == 0 and h * w == p
    log2_p = p.bit_length() - 1

    info = plsc.get_sparse_core_info()
    nw = info.num_cores * info.num_subcores
    ppw = np_total // nw
    ksub = 64

    table = jnp.transpose(sample_map, (0, 2, 3, 1)).reshape(n * h * w, c)
    pts = sample_pts.reshape(-1)

    mesh = plsc.VectorSubcoreMesh(core_axis_name="c", subcore_axis_name="s")
    body = functools.partial(_sampler_body, ppw, ksub, h, w, c, log2_p)

    def wrapped(table_hbm, pts_hbm, out_hbm, pts_v,
                i000, i001, i010, i011, w000, w001, w010, w011,
                i100, i101, i110, i111, w100, w101, w110, w111,
                r000, r001, r010, r011, r100, r101, r110, r111,
                out_v, gsem0, gsem1):
        iw = (((i000, i001, i010, i011), (w000, w001, w010, w011)),
              ((i100, i101, i110, i111), (w100, w101, w110, w111)))
        rbuf = ((r000, r001, r010, r011), (r100, r101, r110, r111))
        body(table_hbm, pts_hbm, out_hbm, pts_v, iw, rbuf,
             out_v, (gsem0, gsem1))

    ivecs = [pltpu.VMEM((ksub,), jnp.int32)] * 4
    wvecs = [pltpu.VMEM((ksub,), jnp.float32)] * 4
    rvecs = [pltpu.VMEM((ksub, c), jnp.float32)] * 4
    out = pl.kernel(
        wrapped,
        out_type=jax.ShapeDtypeStruct((np_total, c), jnp.float32),
        mesh=mesh,
        compiler_params=pltpu.CompilerParams(
            needs_layout_passes=False, use_tc_tiling_on_sc=False),
        scratch_types=(
            [pltpu.VMEM((2 * ppw,), jnp.float32)]
            + ivecs + wvecs + ivecs + wvecs + rvecs + rvecs
            + [pltpu.VMEM((ksub, c), jnp.float32)]
            + [pltpu.SemaphoreType.DMA] * 2
        ),
    )(table, pts)
    return out.reshape(n, p, c)


# speedup vs baseline: 2.8756x; 2.8756x over previous
import functools

import jax
import jax.numpy as jnp
from jax import lax
from jax.experimental import pallas as pl
from jax.experimental.pallas import tpu as pltpu
from jax.experimental.pallas import tpu_sc as plsc

L = 16


def _floor_f32(x):
    t = x.astype(jnp.int32)
    return t - jnp.where(t.astype(jnp.float32) > x, 1, 0)


def _sampler_body(ppw, ksub, h, w, c, log2_p,
                  table_hbm, pts_hbm, out_hbm,
                  pts_v, iw, rbuf, out_v, gsems):
    num_cores = plsc.get_sparse_core_info().num_cores
    wid = lax.axis_index("s") * num_cores + lax.axis_index("c")
    pbase = wid * ppw
    pltpu.sync_copy(pts_hbm.at[pl.ds(2 * pbase, 2 * ppw)], pts_v)

    nsub = ppw // ksub
    lanes = lax.iota(jnp.int32, L)
    lanes2 = lanes * 2

    def stage(j, slot):
        i_v = iw[slot][0]
        w_v = iw[slot][1]
        for g in range(ksub // L):
            off = j * ksub + g * L
            px = plsc.load_gather(pts_v, [lanes2 + 2 * off])
            py = plsc.load_gather(pts_v, [lanes2 + (2 * off + 1)])
            gx = (2.0 * px - 1.0) + 1.0
            gy = (2.0 * py - 1.0) + 1.0
            x = (gx * w - 1.0) * 0.5
            y = (gy * h - 1.0) * 0.5
            x0 = _floor_f32(x)
            y0 = _floor_f32(y)
            wx1 = x - x0.astype(jnp.float32)
            wx0 = 1.0 - wx1
            wy1 = y - y0.astype(jnp.float32)
            wy0 = 1.0 - wy1
            x1 = x0 + 1
            y1 = y0 + 1
            fx0 = jnp.where((x0 >= 0) & (x0 <= w - 1), wx0, 0.0)
            fx1 = jnp.where((x1 >= 0) & (x1 <= w - 1), wx1, 0.0)
            fy0 = jnp.where((y0 >= 0) & (y0 <= h - 1), wy0, 0.0)
            fy1 = jnp.where((y1 >= 0) & (y1 <= h - 1), wy1, 0.0)
            cx0 = jnp.minimum(jnp.maximum(x0, 0), w - 1)
            cx1 = jnp.minimum(jnp.maximum(x1, 0), w - 1)
            cy0 = jnp.minimum(jnp.maximum(y0, 0), h - 1)
            cy1 = jnp.minimum(jnp.maximum(y1, 0), h - 1)
            gp = pbase + off + lanes
            tb = gp & jnp.int32(~(2 ** log2_p - 1))
            row0 = tb + cy0 * w
            row1 = tb + cy1 * w
            sl = pl.ds(g * L, L)
            i_v[0][sl] = row0 + cx0
            i_v[1][sl] = row0 + cx1
            i_v[2][sl] = row1 + cx0
            i_v[3][sl] = row1 + cx1
            w_v[0][sl] = fy0 * fx0
            w_v[1][sl] = fy0 * fx1
            w_v[2][sl] = fy1 * fx0
            w_v[3][sl] = fy1 * fx1
        for k in range(4):
            pltpu.async_copy(table_hbm.at[i_v[k]], rbuf[slot][k], gsems[slot])

    def finish(j, slot):
        i_v = iw[slot][0]
        w_v = iw[slot][1]
        for k in range(4):
            pltpu.make_async_copy(table_hbm.at[i_v[k]], rbuf[slot][k],
                                  gsems[slot]).wait()

        @plsc.parallel_loop(0, ksub, unroll=2)
        def _combine(pt):
            idxv = jnp.full((L,), pt, jnp.int32)
            a00 = plsc.load_gather(w_v[0], [idxv])
            a01 = plsc.load_gather(w_v[1], [idxv])
            a10 = plsc.load_gather(w_v[2], [idxv])
            a11 = plsc.load_gather(w_v[3], [idxv])
            r00, r01, r10, r11 = rbuf[slot]
            for cg in range(c // L):
                slc = pl.ds(cg * L, L)
                val = ((r00[pt, slc] * a00 + r01[pt, slc] * a01)
                       + (r10[pt, slc] * a10 + r11[pt, slc] * a11))
                out_v[pt, slc] = val
        pltpu.sync_copy(out_v, out_hbm.at[pl.ds(pbase + j * ksub, ksub)])

    stage(0, 0)

    def sub2(j2, _):
        j = j2 * 2
        stage(j + 1, 1)
        finish(j, 0)

        @pl.when(j + 2 < nsub)
        def _():
            stage(j + 2, 0)

        finish(j + 1, 1)
        return 0

    lax.fori_loop(0, nsub // 2, sub2, 0)


def kernel(sample_map, sample_pts):
    n, c, h, w = sample_map.shape
    _, p, _ = sample_pts.shape
    np_total = n * p
    assert p & (p - 1) == 0 and h * w == p
    log2_p = p.bit_length() - 1

    info = plsc.get_sparse_core_info()
    nw = info.num_cores * info.num_subcores
    ppw = np_total // nw
    ksub = 64

    table = jnp.transpose(sample_map, (0, 2, 3, 1)).reshape(n * h * w, c)
    pts = sample_pts.reshape(-1)

    mesh = plsc.VectorSubcoreMesh(core_axis_name="c", subcore_axis_name="s")
    body = functools.partial(_sampler_body, ppw, ksub, h, w, c, log2_p)

    def wrapped(table_hbm, pts_hbm, out_hbm, pts_v,
                i000, i001, i010, i011, w000, w001, w010, w011,
                i100, i101, i110, i111, w100, w101, w110, w111,
                r000, r001, r010, r011, r100, r101, r110, r111,
                out_v, gsem0, gsem1):
        iw = (((i000, i001, i010, i011), (w000, w001, w010, w011)),
              ((i100, i101, i110, i111), (w100, w101, w110, w111)))
        rbuf = ((r000, r001, r010, r011), (r100, r101, r110, r111))
        body(table_hbm, pts_hbm, out_hbm, pts_v, iw, rbuf,
             out_v, (gsem0, gsem1))

    ivecs = [pltpu.VMEM((ksub,), jnp.int32)] * 4
    wvecs = [pltpu.VMEM((ksub,), jnp.float32)] * 4
    rvecs = [pltpu.VMEM((ksub, c), jnp.float32)] * 4
    out = pl.kernel(
        wrapped,
        out_type=jax.ShapeDtypeStruct((np_total, c), jnp.float32),
        mesh=mesh,
        compiler_params=pltpu.CompilerParams(
            needs_layout_passes=False, use_tc_tiling_on_sc=False),
        scratch_types=(
            [pltpu.VMEM((2 * ppw,), jnp.float32)]
            + ivecs + wvecs + ivecs + wvecs + rvecs + rvecs
            + [pltpu.VMEM((ksub, c), jnp.float32)]
            + [pltpu.SemaphoreType.DMA] * 2
        ),
    )(table, pts)
    return out.reshape(n, p, c)
